# Initial kernel scaffold; baseline (speedup 1.0000x reference)
import jax
import jax.numpy as jnp
from jax.experimental import pallas as pl


def kernel(x, edge_index, W1, b1, W2, b2, W3, b3, fc1_W, fc1_b, fc2_W, fc2_b):
    raise NotImplementedError("write your pallas kernel here")


# speedup vs baseline: 1.0459x; 1.0459x over previous
import functools

import jax
import jax.numpy as jnp
from jax.experimental import pallas as pl
from jax.experimental.pallas import tpu as pltpu

N_NODES = 100000
ROW_BLOCK = 1000


def _fc_head_body(h_ref, w1_ref, b1_ref, w2_ref, b2_ref, o_ref):
    h = h_ref[...]
    z = jnp.maximum(jnp.dot(h, w1_ref[...], preferred_element_type=jnp.float32)
                    + b1_ref[...][None, :], 0.0)
    z2 = jnp.dot(z, w2_ref[...], preferred_element_type=jnp.float32) + b2_ref[...][None, :]
    o_ref[...] = jax.nn.softmax(z2, axis=-1)


def _fc_head(h, fc1_W, fc1_b, fc2_W, fc2_b):
    n = h.shape[0]
    grid = (n // ROW_BLOCK,)
    return pl.pallas_call(
        _fc_head_body,
        grid=grid,
        in_specs=[
            pl.BlockSpec((ROW_BLOCK, h.shape[1]), lambda i: (i, 0)),
            pl.BlockSpec(fc1_W.shape, lambda i: (0, 0)),
            pl.BlockSpec(fc1_b.shape, lambda i: (0,)),
            pl.BlockSpec(fc2_W.shape, lambda i: (0, 0)),
            pl.BlockSpec(fc2_b.shape, lambda i: (0,)),
        ],
        out_specs=pl.BlockSpec((ROW_BLOCK, 1), lambda i: (i, 0)),
        out_shape=jax.ShapeDtypeStruct((n, 1), jnp.float32),
    )(h, fc1_W, fc1_b, fc2_W, fc2_b)


def _gcn_conv(x, src, dst, W, b, dinv):
    h = x @ W
    norm = dinv[src] * dinv[dst]
    msg = jnp.take(h, src, axis=0) * norm[:, None]
    agg = jnp.zeros_like(h).at[dst].add(msg)
    agg = agg + h * (dinv * dinv)[:, None]
    return agg + b


def kernel(x, edge_index, W1, b1, W2, b2, W3, b3, fc1_W, fc1_b, fc2_W, fc2_b):
    src = edge_index[0]
    dst = edge_index[1]
    n = x.shape[0]
    ones = jnp.ones((src.shape[0],), jnp.float32)
    deg = jnp.ones((n,), jnp.float32).at[dst].add(ones)
    dinv = jax.lax.rsqrt(deg)
    h = jax.nn.relu(_gcn_conv(x, src, dst, W1, b1, dinv))
    h = jax.nn.relu(_gcn_conv(h, src, dst, W2, b2, dinv))
    h = jax.nn.relu(_gcn_conv(h, src, dst, W3, b3, dinv))
    return _fc_head(h, fc1_W, fc1_b, fc2_W, fc2_b)


# speedup vs baseline: 20.7941x; 19.8823x over previous
import functools

import jax
import jax.numpy as jnp
from jax import lax
from jax.experimental import pallas as pl
from jax.experimental.pallas import tpu as pltpu
from jax.experimental.pallas import tpu_sc as plsc

N_NODES = 100000
N_EDGES = 1600000
HALF = 50000
TILES = 16
ACC_ROWS = HALF + TILES
EROWS_PER_TILE = 784
EROWS = TILES * EROWS_PER_TILE
E_PAD = EROWS * 128
CHUNK_ROWS = 16
N_CHUNKS = EROWS_PER_TILE // CHUNK_ROWS
ZROWS = 3200
ZTAIL = ACC_ROWS - 15 * ZROWS
OTAIL = HALF - 15 * ZROWS
F = 40
BLK = 2000

_sc_mesh = plsc.VectorSubcoreMesh(core_axis_name="c", subcore_axis_name="s")
_sc_params = pltpu.CompilerParams(use_tc_tiling_on_sc=False)


def _remap_dst(dstv, node_base, dummy):
    for r in range(CHUNK_ROWS):
        for j in range(128 // 16):
            d = dstv[r, pl.ds(j * 16, 16)]
            local = d - node_base
            m = (local >= 0) & (local < HALF)
            dstv[r, pl.ds(j * 16, 16)] = jnp.where(m, local, dummy)


@functools.partial(
    pl.kernel,
    mesh=_sc_mesh,
    out_type=jax.ShapeDtypeStruct((N_NODES, F), jnp.bfloat16),
    scratch_types=[
        pltpu.VMEM((CHUNK_ROWS, 128), jnp.int32),
        pltpu.VMEM((CHUNK_ROWS, 128), jnp.int32),
        pltpu.VMEM((CHUNK_ROWS * 128, F), jnp.bfloat16),
        pltpu.VMEM_SHARED((ACC_ROWS, F), jnp.bfloat16),
        pltpu.SemaphoreType.DMA,
        pltpu.SemaphoreType.DMA,
    ],
    compiler_params=_sc_params,
)
def _sc_scatter(src_hbm, dst_hbm, xs_hbm, zeros_hbm, out_hbm,
                srcv, dstv, rows, acc, gsem, ssem):
    cid = lax.axis_index("c")
    sid = lax.axis_index("s")
    node_base = cid * HALF
    dummy = HALF + sid

    @pl.when(sid < TILES - 1)
    def _():
        pltpu.sync_copy(zeros_hbm, acc.at[pl.ds(sid * ZROWS, ZROWS)])

    @pl.when(sid == TILES - 1)
    def _():
        pltpu.sync_copy(zeros_hbm.at[pl.ds(0, ZTAIL)],
                        acc.at[pl.ds(15 * ZROWS, ZTAIL)])

    plsc.subcore_barrier()

    def chunk_body(c, carry):
        rb = sid * EROWS_PER_TILE + c * CHUNK_ROWS
        pltpu.sync_copy(src_hbm.at[pl.ds(rb, CHUNK_ROWS)], srcv)
        pltpu.sync_copy(dst_hbm.at[pl.ds(rb, CHUNK_ROWS)], dstv)
        _remap_dst(dstv, node_base, dummy)
        g = [pltpu.async_copy(xs_hbm.at[srcv.at[r]],
                              rows.at[pl.ds(r * 128, 128)], gsem)
             for r in range(CHUNK_ROWS)]
        s = []
        for r in range(CHUNK_ROWS):
            g[r].wait()
            s.append(pltpu.async_copy(rows.at[pl.ds(r * 128, 128)],
                                      acc.at[dstv.at[r]], ssem, add=True))
        for h in s:
            h.wait()
        return carry

    lax.fori_loop(0, N_CHUNKS, chunk_body, 0)
    plsc.subcore_barrier()

    @pl.when(sid < TILES - 1)
    def _():
        pltpu.sync_copy(acc.at[pl.ds(sid * ZROWS, ZROWS)],
                        out_hbm.at[pl.ds(node_base + sid * ZROWS, ZROWS)])

    @pl.when(sid == TILES - 1)
    def _():
        pltpu.sync_copy(acc.at[pl.ds(15 * ZROWS, OTAIL)],
                        out_hbm.at[pl.ds(node_base + 15 * ZROWS, OTAIL)])


@functools.partial(
    pl.kernel,
    mesh=_sc_mesh,
    out_type=jax.ShapeDtypeStruct((N_NODES, 1), jnp.float32),
    scratch_types=[
        pltpu.VMEM((CHUNK_ROWS, 128), jnp.int32),
        pltpu.VMEM((128, 1), jnp.float32),
        pltpu.VMEM_SHARED((ACC_ROWS, 1), jnp.float32),
        pltpu.SemaphoreType.DMA,
    ],
    compiler_params=_sc_params,
)
def _sc_degree(dst_hbm, ones_hbm, zeros_hbm, out_hbm, dstv, ones_v, acc, ssem):
    cid = lax.axis_index("c")
    sid = lax.axis_index("s")
    node_base = cid * HALF
    dummy = HALF + sid

    pltpu.sync_copy(ones_hbm, ones_v)

    @pl.when(sid < TILES - 1)
    def _():
        pltpu.sync_copy(zeros_hbm, acc.at[pl.ds(sid * ZROWS, ZROWS)])

    @pl.when(sid == TILES - 1)
    def _():
        pltpu.sync_copy(zeros_hbm.at[pl.ds(0, ZTAIL)],
                        acc.at[pl.ds(15 * ZROWS, ZTAIL)])

    plsc.subcore_barrier()

    def chunk_body(c, carry):
        rb = sid * EROWS_PER_TILE + c * CHUNK_ROWS
        pltpu.sync_copy(dst_hbm.at[pl.ds(rb, CHUNK_ROWS)], dstv)
        _remap_dst(dstv, node_base, dummy)
        s = [pltpu.async_copy(ones_v, acc.at[dstv.at[r]], ssem, add=True)
             for r in range(CHUNK_ROWS)]
        for h in s:
            h.wait()
        return carry

    lax.fori_loop(0, N_CHUNKS, chunk_body, 0)
    plsc.subcore_barrier()

    @pl.when(sid < TILES - 1)
    def _():
        pltpu.sync_copy(acc.at[pl.ds(sid * ZROWS, ZROWS)],
                        out_hbm.at[pl.ds(node_base + sid * ZROWS, ZROWS)])

    @pl.when(sid == TILES - 1)
    def _():
        pltpu.sync_copy(acc.at[pl.ds(15 * ZROWS, OTAIL)],
                        out_hbm.at[pl.ds(node_base + 15 * ZROWS, OTAIL)])


def _pre_body(cnt_ref, x_ref, dinv_ref, xs_ref):
    d = lax.rsqrt(cnt_ref[...] + 1.0)
    dinv_ref[...] = d
    xs_ref[...] = (x_ref[...] * d).astype(jnp.bfloat16)


def _tc_pre(counts, x):
    grid = (N_NODES // BLK,)
    return pl.pallas_call(
        _pre_body,
        grid=grid,
        in_specs=[
            pl.BlockSpec((BLK, 1), lambda i: (i, 0)),
            pl.BlockSpec((BLK, F), lambda i: (i, 0)),
        ],
        out_specs=[
            pl.BlockSpec((BLK, 1), lambda i: (i, 0)),
            pl.BlockSpec((BLK, F), lambda i: (i, 0)),
        ],
        out_shape=[
            jax.ShapeDtypeStruct((N_NODES, 1), jnp.float32),
            jax.ShapeDtypeStruct((N_NODES, F), jnp.bfloat16),
        ],
    )(counts, x)


def _layer_body(split, agg_ref, xs_ref, dinv_ref, w_ref, b_ref, *out_refs):
    d = dinv_ref[...]
    t = (agg_ref[...].astype(jnp.float32)
         + xs_ref[...].astype(jnp.float32)) * d
    h = jnp.dot(t, w_ref[...], preferred_element_type=jnp.float32)
    z = (jnp.maximum(h + b_ref[...][None, :], 0.0) * d).astype(jnp.bfloat16)
    if split:
        out_refs[0][...] = z[:, :F]
        out_refs[1][...] = z[:, F:]
    else:
        out_refs[0][...] = z


def _tc_layer(agg, xs, dinv, W, b, split):
    fo = W.shape[1]
    grid = (N_NODES // BLK,)
    if split:
        out_specs = [pl.BlockSpec((BLK, F), lambda i: (i, 0))] * 2
        out_shape = [jax.ShapeDtypeStruct((N_NODES, F), jnp.bfloat16)] * 2
    else:
        out_specs = [pl.BlockSpec((BLK, fo), lambda i: (i, 0))]
        out_shape = [jax.ShapeDtypeStruct((N_NODES, fo), jnp.bfloat16)]
    return pl.pallas_call(
        functools.partial(_layer_body, split),
        grid=grid,
        in_specs=[
            pl.BlockSpec((BLK, W.shape[0]), lambda i: (i, 0)),
            pl.BlockSpec((BLK, W.shape[0]), lambda i: (i, 0)),
            pl.BlockSpec((BLK, 1), lambda i: (i, 0)),
            pl.BlockSpec(W.shape, lambda i: (0, 0)),
            pl.BlockSpec(b.shape, lambda i: (0,)),
        ],
        out_specs=out_specs,
        out_shape=out_shape,
    )(agg, xs, dinv, W, b)


def _head_body(a3a_ref, a3b_ref, x3a_ref, x3b_ref, dinv_ref,
               w3_ref, b3_ref, f1w_ref, f1b_ref, f2w_ref, f2b_ref, o_ref):
    d = dinv_ref[...]
    t = jnp.concatenate(
        [(a3a_ref[...].astype(jnp.float32)
          + x3a_ref[...].astype(jnp.float32)) * d,
         (a3b_ref[...].astype(jnp.float32)
          + x3b_ref[...].astype(jnp.float32)) * d], axis=1)
    h3 = jnp.maximum(
        jnp.dot(t, w3_ref[...], preferred_element_type=jnp.float32)
        + b3_ref[...][None, :], 0.0)
    z = jnp.maximum(
        jnp.dot(h3, f1w_ref[...], preferred_element_type=jnp.float32)
        + f1b_ref[...][None, :], 0.0)
    y = jnp.dot(z, f2w_ref[...], preferred_element_type=jnp.float32) \
        + f2b_ref[...][None, :]
    o_ref[...] = jax.nn.softmax(y, axis=-1)


def _tc_head(a3a, a3b, x3a, x3b, dinv, W3, b3, fc1_W, fc1_b, fc2_W, fc2_b):
    grid = (N_NODES // BLK,)
    row = lambda i: (i, 0)
    rep2 = lambda i: (0, 0)
    rep1 = lambda i: (0,)
    return pl.pallas_call(
        _head_body,
        grid=grid,
        in_specs=[
            pl.BlockSpec((BLK, F), row),
            pl.BlockSpec((BLK, F), row),
            pl.BlockSpec((BLK, F), row),
            pl.BlockSpec((BLK, F), row),
            pl.BlockSpec((BLK, 1), row),
            pl.BlockSpec(W3.shape, rep2),
            pl.BlockSpec(b3.shape, rep1),
            pl.BlockSpec(fc1_W.shape, rep2),
            pl.BlockSpec(fc1_b.shape, rep1),
            pl.BlockSpec(fc2_W.shape, rep2),
            pl.BlockSpec(fc2_b.shape, rep1),
        ],
        out_specs=pl.BlockSpec((BLK, 1), row),
        out_shape=jax.ShapeDtypeStruct((N_NODES, 1), jnp.float32),
    )(a3a, a3b, x3a, x3b, dinv, W3, b3, fc1_W, fc1_b, fc2_W, fc2_b)


def kernel(x, edge_index, W1, b1, W2, b2, W3, b3, fc1_W, fc1_b, fc2_W, fc2_b):
    src = edge_index[0].astype(jnp.int32)
    dst = edge_index[1].astype(jnp.int32)
    pad = E_PAD - N_EDGES
    src_p = jnp.concatenate(
        [src, jnp.zeros((pad,), jnp.int32)]).reshape(EROWS, 128)
    dst_p = jnp.concatenate(
        [dst, jnp.full((pad,), jnp.int32(2 ** 30))]).reshape(EROWS, 128)
    zeros40 = jnp.zeros((ZROWS, F), jnp.bfloat16)
    zeros1 = jnp.zeros((ZROWS, 1), jnp.float32)
    ones1 = jnp.ones((128, 1), jnp.float32)

    counts = _sc_degree(dst_p, ones1, zeros1)
    dinv, xs1 = _tc_pre(counts, x)
    agg1 = _sc_scatter(src_p, dst_p, xs1, zeros40)
    (xs2,) = _tc_layer(agg1, xs1, dinv, W1, b1, split=False)
    agg2 = _sc_scatter(src_p, dst_p, xs2, zeros40)
    xs3a, xs3b = _tc_layer(agg2, xs2, dinv, W2, b2, split=True)
    agg3a = _sc_scatter(src_p, dst_p, xs3a, zeros40)
    agg3b = _sc_scatter(src_p, dst_p, xs3b, zeros40)
    return _tc_head(agg3a, agg3b, xs3a, xs3b, dinv,
                    W3, b3, fc1_W, fc1_b, fc2_W, fc2_b)
